# Initial kernel scaffold; baseline (speedup 1.0000x reference)
import jax
import jax.numpy as jnp
from jax.experimental import pallas as pl


def kernel(pixel_values, params):
    raise NotImplementedError("write your pallas kernel here")


# speedup vs baseline: 1.2028x; 1.2028x over previous
import math

import jax
import jax.numpy as jnp
from jax.experimental import pallas as pl

D = 768
I = 3072
H = 12
DH = 64
E = 8
NC = 100
T = 197
TP = 208
EPAD = 128

_bf16 = jnp.bfloat16


def _mm(a, b):
    return jax.lax.dot_general(
        a, b, (((a.ndim - 1,), (0,)), ((), ())),
        preferred_element_type=jnp.float32)


def _ln(x, g, b):
    m = jnp.mean(x, axis=-1, keepdims=True)
    v = jnp.mean((x - m) ** 2, axis=-1, keepdims=True)
    return (x - m) / jnp.sqrt(v + 1e-12) * g + b


def _gelu(x):
    return 0.5 * x * (1.0 + jax.lax.erf(x / math.sqrt(2.0)))


def _embed_kern(p_ref, w_ref, b_ref, o_ref):
    o_ref[:] = _mm(p_ref[:].astype(_bf16), w_ref[:]) + b_ref[:]


def _qkv_kern(x_ref, g_ref, b_ref, w_ref, bias_ref, o_ref):
    h = _ln(x_ref[:], g_ref[:], b_ref[:])
    o_ref[:] = _mm(h.astype(_bf16), w_ref[:]) + bias_ref[:]


def _attn_kern(q_ref, k_ref, v_ref, o_ref):
    valid = jax.lax.broadcasted_iota(jnp.int32, (TP, TP), 1) < T
    for j in (0, 1):
        q = q_ref[:, j * DH:(j + 1) * DH].astype(_bf16)
        k = k_ref[:, j * DH:(j + 1) * DH].astype(_bf16)
        v = v_ref[:, j * DH:(j + 1) * DH].astype(_bf16)
        s = jax.lax.dot_general(
            q, k, (((1,), (1,)), ((), ())),
            preferred_element_type=jnp.float32)
        s = s * (1.0 / math.sqrt(DH))
        s = jnp.where(valid, s, -1e30)
        m = jnp.max(s, axis=-1, keepdims=True)
        p = jnp.exp(s - m)
        a = p / jnp.sum(p, axis=-1, keepdims=True)
        o_ref[:, j * DH:(j + 1) * DH] = _mm(a.astype(_bf16), v)


def _ffn_kern(ctx_ref, x_ref, ow_ref, ob_ref, g_ref, b_ref,
              w1_ref, b1_ref, w2_ref, b2_ref, o_ref):
    x1 = _mm(ctx_ref[:].astype(_bf16), ow_ref[:]) + ob_ref[:] + x_ref[:]
    h2 = _ln(x1, g_ref[:], b_ref[:])
    hid = _gelu(_mm(h2.astype(_bf16), w1_ref[:]) + b1_ref[:])
    o_ref[:] = x1 + _mm(hid.astype(_bf16), w2_ref[:]) + b2_ref[:]


def _router_kern(ctx_ref, x_ref, ow_ref, ob_ref, g_ref, b_ref, gw_ref,
                 x1_ref, h2_ref, cw_ref):
    x1 = _mm(ctx_ref[:].astype(_bf16), ow_ref[:]) + ob_ref[:] + x_ref[:]
    x1_ref[:] = x1
    h2 = _ln(x1, g_ref[:], b_ref[:])
    h2_ref[:] = h2
    n = x1.shape[0]
    logits = _mm(h2.astype(_bf16), gw_ref[:])
    col = jax.lax.broadcasted_iota(jnp.int32, (n, EPAD), 1)
    logits = jnp.where(col < E, logits, -1e30)
    m = jnp.max(logits, axis=-1, keepdims=True)
    p = jnp.exp(logits - m)
    p = p / jnp.sum(p, axis=-1, keepdims=True)
    m1 = jnp.max(p, axis=-1, keepdims=True)
    i1 = jnp.min(jnp.where(p == m1, col, EPAD), axis=-1, keepdims=True)
    p2 = jnp.where(col == i1, -1.0, p)
    m2 = jnp.max(p2, axis=-1, keepdims=True)
    i2 = jnp.min(jnp.where(p2 == m2, col, EPAD), axis=-1, keepdims=True)
    s = m1 + m2 + 1e-9
    cw_ref[:] = (jnp.where(col == i1, m1, 0.0)
                 + jnp.where(col == i2, m2, 0.0)) / s


def _moe_expert_kern(h2_ref, cw_ref, x1_ref, w1_ref, b1_ref, w2_ref, b2_ref,
                     o_ref):
    e = pl.program_id(0)

    @pl.when(e == 0)
    def _():
        o_ref[:] = x1_ref[:]

    hid = _gelu(_mm(h2_ref[:].astype(_bf16), w1_ref[0]) + b1_ref[0])
    ye = _mm(hid.astype(_bf16), w2_ref[0]) + b2_ref[0]
    o_ref[:] += cw_ref[0] * ye


def _head_kern(x_ref, g_ref, b_ref, w_ref, hb_ref, o_ref):
    h = _ln(x_ref[:], g_ref[:], b_ref[:])
    o_ref[:] = _mm(h.astype(_bf16), w_ref[:]) + hb_ref[:]


def _row(v):
    return v.reshape(1, -1)


def kernel(pixel_values, params):
    b = pixel_values.shape[0]
    n = b * TP
    f32 = jnp.float32

    patches = (pixel_values.reshape(b, 3, 14, 16, 14, 16)
               .transpose(0, 2, 4, 1, 3, 5).reshape(b * 196, 768))
    emb = pl.pallas_call(
        _embed_kern,
        out_shape=jax.ShapeDtypeStruct((b * 196, D), f32),
    )(patches, params["patch_w"].astype(_bf16), _row(params["patch_b"]))
    emb = emb.reshape(b, 196, D)
    cls = jnp.broadcast_to(params["cls"], (b, 1, D))
    x = jnp.concatenate([cls, emb], axis=1) + params["pos"]
    x = jnp.pad(x, ((0, 0), (0, TP - T), (0, 0))).reshape(n, D)

    for lp in params["layers"]:
        wqkv = jnp.concatenate(
            [lp["q_w"], lp["k_w"], lp["v_w"]], axis=1).astype(_bf16)
        bqkv = jnp.concatenate(
            [lp["q_b"], lp["k_b"], lp["v_b"]]).reshape(1, 3 * D)
        qkv = pl.pallas_call(
            _qkv_kern,
            out_shape=jax.ShapeDtypeStruct((n, 3 * D), f32),
        )(x, _row(lp["ln1_g"]), _row(lp["ln1_b"]), wqkv, bqkv)

        ctx = pl.pallas_call(
            _attn_kern,
            grid=(b, H // 2),
            in_specs=[
                pl.BlockSpec((TP, 2 * DH), lambda i, h: (i, h)),
                pl.BlockSpec((TP, 2 * DH), lambda i, h: (i, H // 2 + h)),
                pl.BlockSpec((TP, 2 * DH), lambda i, h: (i, H + h)),
            ],
            out_specs=pl.BlockSpec((TP, 2 * DH), lambda i, h: (i, h)),
            out_shape=jax.ShapeDtypeStruct((n, D), f32),
        )(qkv, qkv, qkv)

        if "gate_w" in lp:
            gw = jnp.pad(lp["gate_w"], ((0, 0), (0, EPAD - E))).astype(_bf16)
            x1, h2, cw = pl.pallas_call(
                _router_kern,
                out_shape=[
                    jax.ShapeDtypeStruct((n, D), f32),
                    jax.ShapeDtypeStruct((n, D), f32),
                    jax.ShapeDtypeStruct((n, EPAD), f32),
                ],
            )(ctx, x, lp["o_w"].astype(_bf16), _row(lp["o_b"]),
              _row(lp["ln2_g"]), _row(lp["ln2_b"]), gw)
            cw_t = jnp.transpose(cw[:, :E]).reshape(E, n, 1)
            x = pl.pallas_call(
                _moe_expert_kern,
                grid=(E,),
                in_specs=[
                    pl.BlockSpec((n, D), lambda e: (0, 0)),
                    pl.BlockSpec((1, n, 1), lambda e: (e, 0, 0)),
                    pl.BlockSpec((n, D), lambda e: (0, 0)),
                    pl.BlockSpec((1, D, I), lambda e: (e, 0, 0)),
                    pl.BlockSpec((1, 1, I), lambda e: (e, 0, 0)),
                    pl.BlockSpec((1, I, D), lambda e: (e, 0, 0)),
                    pl.BlockSpec((1, 1, D), lambda e: (e, 0, 0)),
                ],
                out_specs=pl.BlockSpec((n, D), lambda e: (0, 0)),
                out_shape=jax.ShapeDtypeStruct((n, D), f32),
            )(h2, cw_t, x1,
              lp["e_w1"].astype(_bf16), lp["e_b1"].reshape(E, 1, I),
              lp["e_w2"].astype(_bf16), lp["e_b2"].reshape(E, 1, D))
        else:
            x = pl.pallas_call(
                _ffn_kern,
                out_shape=jax.ShapeDtypeStruct((n, D), f32),
            )(ctx, x, lp["o_w"].astype(_bf16), _row(lp["o_b"]),
              _row(lp["ln2_g"]), _row(lp["ln2_b"]),
              lp["w1"].astype(_bf16), _row(lp["b1"]),
              lp["w2"].astype(_bf16), _row(lp["b2"]))

    cls_tok = x.reshape(b, TP, D)[:, 0, :]
    cls_tok = jnp.pad(cls_tok, ((0, 8 - b), (0, 0)))
    hw = jnp.pad(params["head_w"], ((0, 0), (0, 128 - NC))).astype(_bf16)
    hb = jnp.pad(params["head_b"], (0, 128 - NC)).reshape(1, 128)
    logits = pl.pallas_call(
        _head_kern,
        out_shape=jax.ShapeDtypeStruct((8, 128), f32),
    )(cls_tok, _row(params["ln_f_g"]), _row(params["ln_f_b"]), hw, hb)
    return logits[:b, :NC]


# speedup vs baseline: 1.6392x; 1.3629x over previous
import math

import jax
import jax.numpy as jnp
from jax.experimental import pallas as pl

D = 768
I = 3072
H = 12
DH = 64
E = 8
NC = 100
T = 197
TP = 208
EPAD = 128

_bf16 = jnp.bfloat16


def _mm(a, b):
    return jax.lax.dot_general(
        a, b, (((a.ndim - 1,), (0,)), ((), ())),
        preferred_element_type=jnp.float32)


def _ln(x, g, b):
    m = jnp.mean(x, axis=-1, keepdims=True)
    v = jnp.mean((x - m) ** 2, axis=-1, keepdims=True)
    return (x - m) / jnp.sqrt(v + 1e-12) * g + b


def _gelu(x):
    return 0.5 * x * (1.0 + jax.lax.erf(x / math.sqrt(2.0)))


def _embed_kern(p_ref, w_ref, b_ref, o_ref):
    o_ref[:] = _mm(p_ref[:].astype(_bf16), w_ref[:].astype(_bf16)) + b_ref[:]


def _qkv_kern(x_ref, g_ref, b_ref, qw_ref, kw_ref, vw_ref, bias_ref, o_ref):
    h = _ln(x_ref[:], g_ref[:], b_ref[:]).astype(_bf16)
    o_ref[:, 0:D] = _mm(h, qw_ref[:].astype(_bf16)) + bias_ref[:, 0:D]
    o_ref[:, D:2 * D] = _mm(h, kw_ref[:].astype(_bf16)) + bias_ref[:, D:2 * D]
    o_ref[:, 2 * D:3 * D] = (_mm(h, vw_ref[:].astype(_bf16))
                             + bias_ref[:, 2 * D:3 * D])


def _attn_kern(q_ref, k_ref, v_ref, o_ref):
    valid = jax.lax.broadcasted_iota(jnp.int32, (TP, TP), 1) < T
    for j in range(H):
        q = q_ref[:, j * DH:(j + 1) * DH].astype(_bf16)
        k = k_ref[:, j * DH:(j + 1) * DH].astype(_bf16)
        v = v_ref[:, j * DH:(j + 1) * DH].astype(_bf16)
        s = jax.lax.dot_general(
            q, k, (((1,), (1,)), ((), ())),
            preferred_element_type=jnp.float32)
        s = s * (1.0 / math.sqrt(DH))
        s = jnp.where(valid, s, -1e30)
        m = jnp.max(s, axis=-1, keepdims=True)
        p = jnp.exp(s - m)
        a = p / jnp.sum(p, axis=-1, keepdims=True)
        o_ref[:, j * DH:(j + 1) * DH] = _mm(a.astype(_bf16), v)


def _ffn_kern(ctx_ref, x_ref, ow_ref, ob_ref, g_ref, b_ref,
              w1_ref, b1_ref, w2_ref, b2_ref, o_ref):
    x1 = (_mm(ctx_ref[:].astype(_bf16), ow_ref[:].astype(_bf16))
          + ob_ref[:] + x_ref[:])
    h2 = _ln(x1, g_ref[:], b_ref[:])
    hid = _gelu(_mm(h2.astype(_bf16), w1_ref[:].astype(_bf16)) + b1_ref[:])
    o_ref[:] = x1 + _mm(hid.astype(_bf16), w2_ref[:].astype(_bf16)) + b2_ref[:]


def _router_kern(ctx_ref, x_ref, ow_ref, ob_ref, g_ref, b_ref, gw_ref,
                 x1_ref, h2_ref, cw_ref):
    x1 = (_mm(ctx_ref[:].astype(_bf16), ow_ref[:].astype(_bf16))
          + ob_ref[:] + x_ref[:])
    x1_ref[:] = x1
    h2 = _ln(x1, g_ref[:], b_ref[:])
    h2_ref[:] = h2
    n = x1.shape[0]
    logits = _mm(h2.astype(_bf16), gw_ref[:].astype(_bf16))
    col = jax.lax.broadcasted_iota(jnp.int32, (n, EPAD), 1)
    logits = jnp.where(col < E, logits, -1e30)
    m = jnp.max(logits, axis=-1, keepdims=True)
    p = jnp.exp(logits - m)
    p = p / jnp.sum(p, axis=-1, keepdims=True)
    m1 = jnp.max(p, axis=-1, keepdims=True)
    i1 = jnp.min(jnp.where(p == m1, col, EPAD), axis=-1, keepdims=True)
    p2 = jnp.where(col == i1, -1.0, p)
    m2 = jnp.max(p2, axis=-1, keepdims=True)
    i2 = jnp.min(jnp.where(p2 == m2, col, EPAD), axis=-1, keepdims=True)
    s = m1 + m2 + 1e-9
    cw_ref[:] = (jnp.where(col == i1, m1, 0.0)
                 + jnp.where(col == i2, m2, 0.0)) / s


def _moe_expert_kern(h2_ref, cw_ref, x1_ref, w1_ref, b1_ref, w2_ref, b2_ref,
                     o_ref):
    e = pl.program_id(0)

    @pl.when(e == 0)
    def _():
        o_ref[:] = x1_ref[:]

    hid = _gelu(_mm(h2_ref[:].astype(_bf16), w1_ref[0].astype(_bf16))
                + b1_ref[0])
    ye = _mm(hid.astype(_bf16), w2_ref[0].astype(_bf16)) + b2_ref[0]
    o_ref[:] += cw_ref[0] * ye


def _head_kern(x_ref, g_ref, b_ref, w_ref, hb_ref, o_ref):
    h = _ln(x_ref[:], g_ref[:], b_ref[:])
    o_ref[:] = _mm(h.astype(_bf16), w_ref[:].astype(_bf16)) + hb_ref[:]


def _row(v):
    return v.reshape(1, -1)


def kernel(pixel_values, params):
    b = pixel_values.shape[0]
    n = b * TP
    f32 = jnp.float32

    patches = (pixel_values.reshape(b, 3, 14, 16, 14, 16)
               .transpose(0, 2, 4, 1, 3, 5).reshape(b * 196, 768))
    emb = pl.pallas_call(
        _embed_kern,
        out_shape=jax.ShapeDtypeStruct((b * 196, D), f32),
    )(patches, params["patch_w"], _row(params["patch_b"]))
    emb = emb.reshape(b, 196, D)
    cls = jnp.broadcast_to(params["cls"], (b, 1, D))
    x = jnp.concatenate([cls, emb], axis=1) + params["pos"]
    x = jnp.pad(x, ((0, 0), (0, TP - T), (0, 0))).reshape(n, D)

    for lp in params["layers"]:
        bqkv = jnp.concatenate(
            [lp["q_b"], lp["k_b"], lp["v_b"]]).reshape(1, 3 * D)
        qkv = pl.pallas_call(
            _qkv_kern,
            out_shape=jax.ShapeDtypeStruct((n, 3 * D), f32),
        )(x, _row(lp["ln1_g"]), _row(lp["ln1_b"]),
          lp["q_w"], lp["k_w"], lp["v_w"], bqkv)

        ctx = pl.pallas_call(
            _attn_kern,
            grid=(b,),
            in_specs=[
                pl.BlockSpec((TP, D), lambda i: (i, 0)),
                pl.BlockSpec((TP, D), lambda i: (i, 1)),
                pl.BlockSpec((TP, D), lambda i: (i, 2)),
            ],
            out_specs=pl.BlockSpec((TP, D), lambda i: (i, 0)),
            out_shape=jax.ShapeDtypeStruct((n, D), f32),
        )(qkv, qkv, qkv)

        if "gate_w" in lp:
            gw = jnp.pad(lp["gate_w"], ((0, 0), (0, EPAD - E)))
            x1, h2, cw = pl.pallas_call(
                _router_kern,
                out_shape=[
                    jax.ShapeDtypeStruct((n, D), f32),
                    jax.ShapeDtypeStruct((n, D), f32),
                    jax.ShapeDtypeStruct((n, EPAD), f32),
                ],
            )(ctx, x, lp["o_w"], _row(lp["o_b"]),
              _row(lp["ln2_g"]), _row(lp["ln2_b"]), gw)
            cw_t = jnp.transpose(cw[:, :E]).reshape(E, n, 1)
            x = pl.pallas_call(
                _moe_expert_kern,
                grid=(E,),
                in_specs=[
                    pl.BlockSpec((n, D), lambda e: (0, 0)),
                    pl.BlockSpec((1, n, 1), lambda e: (e, 0, 0)),
                    pl.BlockSpec((n, D), lambda e: (0, 0)),
                    pl.BlockSpec((1, D, I), lambda e: (e, 0, 0)),
                    pl.BlockSpec((1, 1, I), lambda e: (e, 0, 0)),
                    pl.BlockSpec((1, I, D), lambda e: (e, 0, 0)),
                    pl.BlockSpec((1, 1, D), lambda e: (e, 0, 0)),
                ],
                out_specs=pl.BlockSpec((n, D), lambda e: (0, 0)),
                out_shape=jax.ShapeDtypeStruct((n, D), f32),
            )(h2, cw_t, x1,
              lp["e_w1"], lp["e_b1"].reshape(E, 1, I),
              lp["e_w2"], lp["e_b2"].reshape(E, 1, D))
        else:
            x = pl.pallas_call(
                _ffn_kern,
                out_shape=jax.ShapeDtypeStruct((n, D), f32),
            )(ctx, x, lp["o_w"], _row(lp["o_b"]),
              _row(lp["ln2_g"]), _row(lp["ln2_b"]),
              lp["w1"].astype(_bf16), _row(lp["b1"]),
              lp["w2"].astype(_bf16), _row(lp["b2"]))

    cls_tok = x.reshape(b, TP, D)[:, 0, :]
    cls_tok = jnp.pad(cls_tok, ((0, 8 - b), (0, 0)))
    hw = jnp.pad(params["head_w"], ((0, 0), (0, 128 - NC)))
    hb = jnp.pad(params["head_b"], (0, 128 - NC)).reshape(1, 128)
    logits = pl.pallas_call(
        _head_kern,
        out_shape=jax.ShapeDtypeStruct((8, 128), f32),
    )(cls_tok, _row(params["ln_f_g"]), _row(params["ln_f_b"]), hw, hb)
    return logits[:b, :NC]


# speedup vs baseline: 2.0414x; 1.2454x over previous
import math

import jax
import jax.numpy as jnp
from jax.experimental import pallas as pl
from jax.experimental.pallas import tpu as pltpu

D = 768
I = 3072
H = 12
DH = 64
E = 8
NC = 100
T = 197
TP = 208
EPAD = 128
NI = 4
IB = I // NI

_bf16 = jnp.bfloat16
f32 = jnp.float32


def _mm(a, b):
    return jax.lax.dot_general(
        a, b, (((a.ndim - 1,), (0,)), ((), ())),
        preferred_element_type=jnp.float32)


def _ln(x, g, b):
    m = jnp.mean(x, axis=-1, keepdims=True)
    v = jnp.mean((x - m) ** 2, axis=-1, keepdims=True)
    return (x - m) / jnp.sqrt(v + 1e-12) * g + b


def _gelu(x):
    return 0.5 * x * (1.0 + jax.lax.erf(x / math.sqrt(2.0)))


def _embed_kern(p_ref, w_ref, b_ref, o_ref):
    o_ref[:] = _mm(p_ref[:].astype(_bf16), w_ref[:].astype(_bf16)) + b_ref[:]


def _attn_body(x_ref, g_ref, b_ref, qw_ref, kw_ref, vw_ref, ow_ref,
               bias_ref, ob_ref, qs, ks, vs, os):
    i = pl.program_id(0)

    @pl.when(i == 0)
    def _():
        qs[:] = qw_ref[:].astype(_bf16)
        ks[:] = kw_ref[:].astype(_bf16)
        vs[:] = vw_ref[:].astype(_bf16)
        os[:] = ow_ref[:].astype(_bf16)

    x = x_ref[:]
    h = _ln(x, g_ref[:], b_ref[:]).astype(_bf16)
    q = _mm(h, qs[:]) + bias_ref[:, 0:D]
    k = _mm(h, ks[:]) + bias_ref[:, D:2 * D]
    v = _mm(h, vs[:]) + bias_ref[:, 2 * D:3 * D]
    valid = jax.lax.broadcasted_iota(jnp.int32, (TP, TP), 1) < T
    parts = []
    for j in range(H):
        qj = q[:, j * DH:(j + 1) * DH].astype(_bf16)
        kj = k[:, j * DH:(j + 1) * DH].astype(_bf16)
        vj = v[:, j * DH:(j + 1) * DH].astype(_bf16)
        s = jax.lax.dot_general(
            qj, kj, (((1,), (1,)), ((), ())),
            preferred_element_type=f32)
        s = s * (1.0 / math.sqrt(DH))
        s = jnp.where(valid, s, -1e30)
        m = jnp.max(s, axis=-1, keepdims=True)
        p = jnp.exp(s - m)
        a = p / jnp.sum(p, axis=-1, keepdims=True)
        parts.append(_mm(a.astype(_bf16), vj))
    ctx = jnp.concatenate(parts, axis=1)
    return _mm(ctx.astype(_bf16), os[:]) + ob_ref[:] + x


def _attn_kern(x_ref, g_ref, b_ref, qw_ref, kw_ref, vw_ref, ow_ref,
               bias_ref, ob_ref, x1_ref, qs, ks, vs, os):
    x1_ref[:] = _attn_body(x_ref, g_ref, b_ref, qw_ref, kw_ref, vw_ref,
                           ow_ref, bias_ref, ob_ref, qs, ks, vs, os)


def _attn_moe_kern(x_ref, g_ref, b_ref, qw_ref, kw_ref, vw_ref, ow_ref,
                   bias_ref, ob_ref, g2_ref, b2_ref, gw_ref,
                   x1_ref, h2_ref, cw_ref, qs, ks, vs, os):
    x1 = _attn_body(x_ref, g_ref, b_ref, qw_ref, kw_ref, vw_ref, ow_ref,
                    bias_ref, ob_ref, qs, ks, vs, os)
    x1_ref[:] = x1
    h2 = _ln(x1, g2_ref[:], b2_ref[:])
    h2_ref[:] = h2
    logits = _mm(h2.astype(_bf16), gw_ref[:].astype(_bf16))
    col = jax.lax.broadcasted_iota(jnp.int32, (TP, EPAD), 1)
    logits = jnp.where(col < E, logits, -1e30)
    m = jnp.max(logits, axis=-1, keepdims=True)
    p = jnp.exp(logits - m)
    p = p / jnp.sum(p, axis=-1, keepdims=True)
    m1 = jnp.max(p, axis=-1, keepdims=True)
    i1 = jnp.min(jnp.where(p == m1, col, EPAD), axis=-1, keepdims=True)
    p2 = jnp.where(col == i1, -1.0, p)
    m2 = jnp.max(p2, axis=-1, keepdims=True)
    i2 = jnp.min(jnp.where(p2 == m2, col, EPAD), axis=-1, keepdims=True)
    s = m1 + m2 + 1e-9
    cw_ref[:] = (jnp.where(col == i1, m1, 0.0)
                 + jnp.where(col == i2, m2, 0.0)) / s


def _ffn_kern(x1_ref, g_ref, b_ref, w1_ref, b1_ref, w2_ref, b2_ref,
              o_ref, h2s):
    j = pl.program_id(0)

    @pl.when(j == 0)
    def _():
        h2s[:] = _ln(x1_ref[:], g_ref[:], b_ref[:])
        o_ref[:] = x1_ref[:] + b2_ref[:]

    hid = _gelu(_mm(h2s[:].astype(_bf16), w1_ref[:].astype(_bf16))
                + b1_ref[:])
    o_ref[:] += _mm(hid.astype(_bf16), w2_ref[:].astype(_bf16))


def _moe_expert_kern(h2_ref, cw_ref, x1_ref, w1_ref, b1_ref, w2_ref, b2_ref,
                     o_ref):
    e = pl.program_id(0)

    @pl.when(e == 0)
    def _():
        o_ref[:] = x1_ref[:]

    hid = _gelu(_mm(h2_ref[:].astype(_bf16), w1_ref[0].astype(_bf16))
                + b1_ref[0])
    ye = _mm(hid.astype(_bf16), w2_ref[0].astype(_bf16)) + b2_ref[0]
    o_ref[:] += cw_ref[0] * ye


def _head_kern(x_ref, g_ref, b_ref, w_ref, hb_ref, o_ref):
    h = _ln(x_ref[:], g_ref[:], b_ref[:])
    o_ref[:] = _mm(h.astype(_bf16), w_ref[:].astype(_bf16)) + hb_ref[:]


def _row(v):
    return v.reshape(1, -1)


def kernel(pixel_values, params):
    b = pixel_values.shape[0]
    n = b * TP

    patches = (pixel_values.reshape(b, 3, 14, 16, 14, 16)
               .transpose(0, 2, 4, 1, 3, 5).reshape(b * 196, 768))
    emb = pl.pallas_call(
        _embed_kern,
        out_shape=jax.ShapeDtypeStruct((b * 196, D), f32),
    )(patches, params["patch_w"], _row(params["patch_b"]))
    emb = emb.reshape(b, 196, D)
    cls = jnp.broadcast_to(params["cls"], (b, 1, D))
    x = jnp.concatenate([cls, emb], axis=1) + params["pos"]
    x = jnp.pad(x, ((0, 0), (0, TP - T), (0, 0))).reshape(n, D)

    wspec = pl.BlockSpec((D, D), lambda i: (0, 0))
    rspec = pl.BlockSpec((1, D), lambda i: (0, 0))
    xspec = pl.BlockSpec((TP, D), lambda i: (i, 0))
    wscratch = [pltpu.VMEM((D, D), _bf16) for _ in range(4)]

    for lp in params["layers"]:
        bqkv = jnp.concatenate(
            [lp["q_b"], lp["k_b"], lp["v_b"]]).reshape(1, 3 * D)
        attn_in = (x, _row(lp["ln1_g"]), _row(lp["ln1_b"]),
                   lp["q_w"], lp["k_w"], lp["v_w"], lp["o_w"],
                   bqkv, _row(lp["o_b"]))
        attn_specs = [xspec, rspec, rspec, wspec, wspec, wspec, wspec,
                      pl.BlockSpec((1, 3 * D), lambda i: (0, 0)), rspec]

        if "gate_w" in lp:
            gw = jnp.pad(lp["gate_w"], ((0, 0), (0, EPAD - E)))
            x1, h2, cw = pl.pallas_call(
                _attn_moe_kern,
                grid=(b,),
                in_specs=attn_specs + [
                    rspec, rspec, pl.BlockSpec((D, EPAD), lambda i: (0, 0))],
                out_specs=[xspec, xspec,
                           pl.BlockSpec((TP, EPAD), lambda i: (i, 0))],
                out_shape=[
                    jax.ShapeDtypeStruct((n, D), f32),
                    jax.ShapeDtypeStruct((n, D), f32),
                    jax.ShapeDtypeStruct((n, EPAD), f32),
                ],
                scratch_shapes=wscratch,
            )(*attn_in, _row(lp["ln2_g"]), _row(lp["ln2_b"]), gw)
            cw_t = jnp.transpose(cw[:, :E]).reshape(E, n, 1)
            x = pl.pallas_call(
                _moe_expert_kern,
                grid=(E,),
                in_specs=[
                    pl.BlockSpec((n, D), lambda e: (0, 0)),
                    pl.BlockSpec((1, n, 1), lambda e: (e, 0, 0)),
                    pl.BlockSpec((n, D), lambda e: (0, 0)),
                    pl.BlockSpec((1, D, I), lambda e: (e, 0, 0)),
                    pl.BlockSpec((1, 1, I), lambda e: (e, 0, 0)),
                    pl.BlockSpec((1, I, D), lambda e: (e, 0, 0)),
                    pl.BlockSpec((1, 1, D), lambda e: (e, 0, 0)),
                ],
                out_specs=pl.BlockSpec((n, D), lambda e: (0, 0)),
                out_shape=jax.ShapeDtypeStruct((n, D), f32),
            )(h2, cw_t, x1,
              lp["e_w1"], lp["e_b1"].reshape(E, 1, I),
              lp["e_w2"], lp["e_b2"].reshape(E, 1, D))
        else:
            x1 = pl.pallas_call(
                _attn_kern,
                grid=(b,),
                in_specs=attn_specs,
                out_specs=xspec,
                out_shape=jax.ShapeDtypeStruct((n, D), f32),
                scratch_shapes=wscratch,
            )(*attn_in)
            x = pl.pallas_call(
                _ffn_kern,
                grid=(NI,),
                in_specs=[
                    pl.BlockSpec((n, D), lambda j: (0, 0)),
                    pl.BlockSpec((1, D), lambda j: (0, 0)),
                    pl.BlockSpec((1, D), lambda j: (0, 0)),
                    pl.BlockSpec((D, IB), lambda j: (0, j)),
                    pl.BlockSpec((1, IB), lambda j: (0, j)),
                    pl.BlockSpec((IB, D), lambda j: (j, 0)),
                    pl.BlockSpec((1, D), lambda j: (0, 0)),
                ],
                out_specs=pl.BlockSpec((n, D), lambda j: (0, 0)),
                out_shape=jax.ShapeDtypeStruct((n, D), f32),
                scratch_shapes=[pltpu.VMEM((n, D), f32)],
            )(x1, _row(lp["ln2_g"]), _row(lp["ln2_b"]),
              lp["w1"], _row(lp["b1"]), lp["w2"], _row(lp["b2"]))

    cls_tok = x.reshape(b, TP, D)[:, 0, :]
    cls_tok = jnp.pad(cls_tok, ((0, 8 - b), (0, 0)))
    hw = jnp.pad(params["head_w"], ((0, 0), (0, 128 - NC)))
    hb = jnp.pad(params["head_b"], (0, 128 - NC)).reshape(1, 128)
    logits = pl.pallas_call(
        _head_kern,
        out_shape=jax.ShapeDtypeStruct((8, 128), f32),
    )(cls_tok, _row(params["ln_f_g"]), _row(params["ln_f_b"]), hw, hb)
    return logits[:b, :NC]


# speedup vs baseline: 2.2118x; 1.0834x over previous
import math

import jax
import jax.numpy as jnp
from jax.experimental import pallas as pl
from jax.experimental.pallas import tpu as pltpu

D = 768
I = 3072
H = 12
DH = 64
E = 8
NC = 100
T = 197
TP = 208
EPAD = 128
NI = 4
IB = I // NI
IH = I // 2

_bf16 = jnp.bfloat16
f32 = jnp.float32


def _mm(a, b):
    return jax.lax.dot_general(
        a, b, (((a.ndim - 1,), (0,)), ((), ())),
        preferred_element_type=jnp.float32)


def _ln(x, g, b):
    m = jnp.mean(x, axis=-1, keepdims=True)
    v = jnp.mean((x - m) ** 2, axis=-1, keepdims=True)
    return (x - m) / jnp.sqrt(v + 1e-12) * g + b


def _gelu(x):
    return 0.5 * x * (1.0 + jax.lax.erf(x / math.sqrt(2.0)))


def _embed_kern(p_ref, w_ref, b_ref, o_ref):
    o_ref[:] = _mm(p_ref[:].astype(_bf16), w_ref[:].astype(_bf16)) + b_ref[:]


def _attn_x1(x_ref, g_ref, b_ref, qw_ref, kw_ref, vw_ref, ow_ref,
             bias_ref, ob_ref, qs, ks, vs, os):
    i = pl.program_id(0)

    @pl.when(i == 0)
    def _():
        qs[:] = qw_ref[:].astype(_bf16)
        ks[:] = kw_ref[:].astype(_bf16)
        vs[:] = vw_ref[:].astype(_bf16)
        os[:] = ow_ref[:].astype(_bf16)

    x = x_ref[:]
    h = _ln(x, g_ref[:], b_ref[:]).astype(_bf16)
    q = _mm(h, qs[:]) + bias_ref[:, 0:D]
    k = _mm(h, ks[:]) + bias_ref[:, D:2 * D]
    v = _mm(h, vs[:]) + bias_ref[:, 2 * D:3 * D]
    valid = jax.lax.broadcasted_iota(jnp.int32, (TP, TP), 1) < T
    parts = []
    for j in range(H):
        qj = q[:, j * DH:(j + 1) * DH].astype(_bf16)
        kj = k[:, j * DH:(j + 1) * DH].astype(_bf16)
        vj = v[:, j * DH:(j + 1) * DH].astype(_bf16)
        s = jax.lax.dot_general(
            qj, kj, (((1,), (1,)), ((), ())),
            preferred_element_type=f32)
        s = s * (1.0 / math.sqrt(DH))
        s = jnp.where(valid, s, -1e30)
        m = jnp.max(s, axis=-1, keepdims=True)
        p = jnp.exp(s - m)
        a = p / jnp.sum(p, axis=-1, keepdims=True)
        parts.append(_mm(a.astype(_bf16), vj))
    ctx = jnp.concatenate(parts, axis=1)
    return _mm(ctx.astype(_bf16), os[:]) + ob_ref[:] + x


def _dense_layer_kern(nb, x_ref, g_ref, b_ref, qw_ref, kw_ref, vw_ref,
                      ow_ref, bias_ref, ob_ref, g2_ref, b2_ref,
                      w1_ref, b1_ref, w2_ref, fb2_ref,
                      o_ref, qs, ks, vs, os, h2s):
    i = pl.program_id(0)

    @pl.when(i < nb)
    def _attn_phase():
        x1 = _attn_x1(x_ref, g_ref, b_ref, qw_ref, kw_ref, vw_ref, ow_ref,
                      bias_ref, ob_ref, qs, ks, vs, os)
        h2s[pl.ds(i * TP, TP), :] = _ln(x1, g2_ref[:], b2_ref[:])
        o_ref[pl.ds(i * TP, TP), :] = x1 + fb2_ref[:]

    @pl.when(i >= nb)
    def _ffn_phase():
        hid = _gelu(_mm(h2s[:].astype(_bf16), w1_ref[:].astype(_bf16))
                    + b1_ref[:])
        o_ref[:] += _mm(hid.astype(_bf16), w2_ref[:].astype(_bf16))


def _moe_layer_kern(nb, x_ref, g_ref, b_ref, qw_ref, kw_ref, vw_ref,
                    ow_ref, bias_ref, ob_ref, g2_ref, b2_ref, gw_ref,
                    w1_ref, b1_ref, w2_ref, eb2_ref,
                    o_ref, qs, ks, vs, os, h2s, cws):
    i = pl.program_id(0)

    @pl.when(i < nb)
    def _attn_phase():
        x1 = _attn_x1(x_ref, g_ref, b_ref, qw_ref, kw_ref, vw_ref, ow_ref,
                      bias_ref, ob_ref, qs, ks, vs, os)
        h2 = _ln(x1, g2_ref[:], b2_ref[:])
        h2s[pl.ds(i * TP, TP), :] = h2
        o_ref[pl.ds(i * TP, TP), :] = x1
        logits = _mm(h2.astype(_bf16), gw_ref[:].astype(_bf16))
        col = jax.lax.broadcasted_iota(jnp.int32, (TP, EPAD), 1)
        logits = jnp.where(col < E, logits, -1e30)
        m = jnp.max(logits, axis=-1, keepdims=True)
        p = jnp.exp(logits - m)
        p = p / jnp.sum(p, axis=-1, keepdims=True)
        m1 = jnp.max(p, axis=-1, keepdims=True)
        i1 = jnp.min(jnp.where(p == m1, col, EPAD), axis=-1, keepdims=True)
        p2 = jnp.where(col == i1, -1.0, p)
        m2 = jnp.max(p2, axis=-1, keepdims=True)
        i2 = jnp.min(jnp.where(p2 == m2, col, EPAD), axis=-1, keepdims=True)
        sw = m1 + m2 + 1e-9
        cws[pl.ds(i * TP, TP), :] = (jnp.where(col == i1, m1, 0.0)
                                     + jnp.where(col == i2, m2, 0.0)) / sw

    @pl.when(i >= nb)
    def _expert_phase():
        e = (i - nb) // 2
        half = (i - nb) % 2
        hid = _gelu(_mm(h2s[:].astype(_bf16), w1_ref[0].astype(_bf16))
                    + b1_ref[0])
        ye = _mm(hid.astype(_bf16), w2_ref[0].astype(_bf16))
        ye = ye + jnp.where(half == 0, 1.0, 0.0) * eb2_ref[0]
        colh = jax.lax.broadcasted_iota(jnp.int32, cws.shape, 1)
        w = jnp.sum(jnp.where(colh == e, cws[:], 0.0), axis=-1, keepdims=True)
        o_ref[:] += w * ye


def _head_kern(x_ref, g_ref, b_ref, w_ref, hb_ref, o_ref):
    h = _ln(x_ref[:], g_ref[:], b_ref[:])
    o_ref[:] = _mm(h.astype(_bf16), w_ref[:].astype(_bf16)) + hb_ref[:]


def _row(v):
    return v.reshape(1, -1)


def kernel(pixel_values, params):
    b = pixel_values.shape[0]
    n = b * TP
    import functools

    patches = (pixel_values.reshape(b, 3, 14, 16, 14, 16)
               .transpose(0, 2, 4, 1, 3, 5).reshape(b * 196, 768))
    emb = pl.pallas_call(
        _embed_kern,
        out_shape=jax.ShapeDtypeStruct((b * 196, D), f32),
    )(patches, params["patch_w"], _row(params["patch_b"]))
    emb = emb.reshape(b, 196, D)
    cls = jnp.broadcast_to(params["cls"], (b, 1, D))
    x = jnp.concatenate([cls, emb], axis=1) + params["pos"]
    x = jnp.pad(x, ((0, 0), (0, TP - T), (0, 0))).reshape(n, D)

    def cst(i):
        return (0, 0)

    wspec = pl.BlockSpec((D, D), cst)
    rspec = pl.BlockSpec((1, D), cst)
    ospec = pl.BlockSpec((n, D), cst)
    wscratch = [pltpu.VMEM((D, D), _bf16) for _ in range(4)]

    for lp in params["layers"]:
        bqkv = jnp.concatenate(
            [lp["q_b"], lp["k_b"], lp["v_b"]]).reshape(1, 3 * D)
        attn_in = (x, _row(lp["ln1_g"]), _row(lp["ln1_b"]),
                   lp["q_w"], lp["k_w"], lp["v_w"], lp["o_w"],
                   bqkv, _row(lp["o_b"]))
        xspec = pl.BlockSpec((TP, D), lambda i: (jnp.minimum(i, b - 1), 0))
        attn_specs = [xspec, rspec, rspec, wspec, wspec, wspec, wspec,
                      pl.BlockSpec((1, 3 * D), cst), rspec]

        if "gate_w" in lp:
            gw = jnp.pad(lp["gate_w"], ((0, 0), (0, EPAD - E)))

            def eidx(i):
                return jnp.maximum(i - b, 0) // 2

            def hidx(i):
                return jnp.maximum(i - b, 0) % 2

            x = pl.pallas_call(
                functools.partial(_moe_layer_kern, b),
                grid=(b + 2 * E,),
                in_specs=attn_specs + [
                    rspec, rspec, pl.BlockSpec((D, EPAD), cst),
                    pl.BlockSpec((1, D, IH), lambda i: (eidx(i), 0, hidx(i))),
                    pl.BlockSpec((1, 1, IH), lambda i: (eidx(i), 0, hidx(i))),
                    pl.BlockSpec((1, IH, D), lambda i: (eidx(i), hidx(i), 0)),
                    pl.BlockSpec((1, 1, D), lambda i: (eidx(i), 0, 0)),
                ],
                out_specs=ospec,
                out_shape=jax.ShapeDtypeStruct((n, D), f32),
                scratch_shapes=wscratch + [pltpu.VMEM((n, D), f32),
                                           pltpu.VMEM((n, EPAD), f32)],
            )(*attn_in, _row(lp["ln2_g"]), _row(lp["ln2_b"]), gw,
              lp["e_w1"], lp["e_b1"].reshape(E, 1, I),
              lp["e_w2"], lp["e_b2"].reshape(E, 1, D))
        else:
            def jidx(i):
                return jnp.maximum(i - b, 0)

            x = pl.pallas_call(
                functools.partial(_dense_layer_kern, b),
                grid=(b + NI,),
                in_specs=attn_specs + [
                    rspec, rspec,
                    pl.BlockSpec((D, IB), lambda i: (0, jidx(i))),
                    pl.BlockSpec((1, IB), lambda i: (0, jidx(i))),
                    pl.BlockSpec((IB, D), lambda i: (jidx(i), 0)),
                    rspec,
                ],
                out_specs=ospec,
                out_shape=jax.ShapeDtypeStruct((n, D), f32),
                scratch_shapes=wscratch + [pltpu.VMEM((n, D), f32)],
            )(*attn_in, _row(lp["ln2_g"]), _row(lp["ln2_b"]),
              lp["w1"], _row(lp["b1"]), lp["w2"], _row(lp["b2"]))

    cls_tok = x.reshape(b, TP, D)[:, 0, :]
    cls_tok = jnp.pad(cls_tok, ((0, 8 - b), (0, 0)))
    hw = jnp.pad(params["head_w"], ((0, 0), (0, 128 - NC)))
    hb = jnp.pad(params["head_b"], (0, 128 - NC)).reshape(1, 128)
    logits = pl.pallas_call(
        _head_kern,
        out_shape=jax.ShapeDtypeStruct((8, 128), f32),
    )(cls_tok, _row(params["ln_f_g"]), _row(params["ln_f_b"]), hw, hb)
    return logits[:b, :NC]


# speedup vs baseline: 2.3143x; 1.0464x over previous
import functools
import math

import jax
import jax.numpy as jnp
from jax.experimental import pallas as pl
from jax.experimental.pallas import tpu as pltpu

D = 768
I = 3072
H = 12
DH = 64
E = 8
NC = 100
T = 197
TP = 208
EPAD = 128
NI = 4
IB = I // NI
NQ = 4
IQ = I // NQ

_bf16 = jnp.bfloat16
f32 = jnp.float32


def _mm(a, b):
    return jax.lax.dot_general(
        a, b, (((a.ndim - 1,), (0,)), ((), ())),
        preferred_element_type=jnp.float32)


def _ln(x, g, b):
    m = jnp.mean(x, axis=-1, keepdims=True)
    v = jnp.mean((x - m) ** 2, axis=-1, keepdims=True)
    return (x - m) / jnp.sqrt(v + 1e-12) * g + b


def _gelu(x):
    return 0.5 * x * (1.0 + jax.lax.erf(x / math.sqrt(2.0)))


def _embed_kern(p_ref, w_ref, b_ref, o_ref):
    o_ref[:] = _mm(p_ref[:].astype(_bf16), w_ref[:].astype(_bf16)) + b_ref[:]


def _attn_x1(nb, x_ref, g_ref, b_ref, qw_ref, kw_ref, vw_ref, ow_ref,
             bias_ref, ob_ref, qs, ks, vs, os):
    qs[:] = qw_ref[:].astype(_bf16)
    ks[:] = kw_ref[:].astype(_bf16)
    vs[:] = vw_ref[:].astype(_bf16)
    os[:] = ow_ref[:].astype(_bf16)
    x = x_ref[:]
    h = _ln(x, g_ref[:], b_ref[:]).astype(_bf16)
    q = _mm(h, qs[:]) + bias_ref[:, 0:D]
    k = _mm(h, ks[:]) + bias_ref[:, D:2 * D]
    v = _mm(h, vs[:]) + bias_ref[:, 2 * D:3 * D]
    valid = jax.lax.broadcasted_iota(jnp.int32, (TP, TP), 1) < T
    rows = []
    for bi in range(nb):
        r0, r1 = bi * TP, (bi + 1) * TP
        parts = []
        for j in range(H):
            c0, c1 = j * DH, (j + 1) * DH
            qj = q[r0:r1, c0:c1].astype(_bf16)
            kj = k[r0:r1, c0:c1].astype(_bf16)
            vj = v[r0:r1, c0:c1].astype(_bf16)
            s = jax.lax.dot_general(
                qj, kj, (((1,), (1,)), ((), ())),
                preferred_element_type=f32)
            s = s * (1.0 / math.sqrt(DH))
            s = jnp.where(valid, s, -1e30)
            m = jnp.max(s, axis=-1, keepdims=True)
            p = jnp.exp(s - m)
            a = p / jnp.sum(p, axis=-1, keepdims=True)
            parts.append(_mm(a.astype(_bf16), vj))
        rows.append(jnp.concatenate(parts, axis=1))
    ctx = jnp.concatenate(rows, axis=0)
    return _mm(ctx.astype(_bf16), os[:]) + ob_ref[:] + x


def _dense_layer_kern(nb, x_ref, g_ref, b_ref, qw_ref, kw_ref, vw_ref,
                      ow_ref, bias_ref, ob_ref, g2_ref, b2_ref,
                      w1_ref, b1_ref, w2_ref, fb2_ref,
                      o_ref, qs, ks, vs, os, h2s):
    i = pl.program_id(0)

    @pl.when(i == 0)
    def _attn_phase():
        x1 = _attn_x1(nb, x_ref, g_ref, b_ref, qw_ref, kw_ref, vw_ref,
                      ow_ref, bias_ref, ob_ref, qs, ks, vs, os)
        h2s[:] = _ln(x1, g2_ref[:], b2_ref[:])
        o_ref[:] = x1 + fb2_ref[:]

    @pl.when(i > 0)
    def _ffn_phase():
        hid = _gelu(_mm(h2s[:].astype(_bf16), w1_ref[:].astype(_bf16))
                    + b1_ref[:])
        o_ref[:] += _mm(hid.astype(_bf16), w2_ref[:].astype(_bf16))


def _moe_layer_kern(nb, x_ref, g_ref, b_ref, qw_ref, kw_ref, vw_ref,
                    ow_ref, bias_ref, ob_ref, g2_ref, b2_ref, gw_ref,
                    w1_ref, b1_ref, w2_ref, eb2_ref,
                    o_ref, qs, ks, vs, os, h2s, cws):
    i = pl.program_id(0)

    @pl.when(i == 0)
    def _attn_phase():
        x1 = _attn_x1(nb, x_ref, g_ref, b_ref, qw_ref, kw_ref, vw_ref,
                      ow_ref, bias_ref, ob_ref, qs, ks, vs, os)
        h2 = _ln(x1, g2_ref[:], b2_ref[:])
        h2s[:] = h2
        o_ref[:] = x1
        n = x1.shape[0]
        logits = _mm(h2.astype(_bf16), gw_ref[:].astype(_bf16))
        col = jax.lax.broadcasted_iota(jnp.int32, (n, EPAD), 1)
        logits = jnp.where(col < E, logits, -1e30)
        m = jnp.max(logits, axis=-1, keepdims=True)
        p = jnp.exp(logits - m)
        p = p / jnp.sum(p, axis=-1, keepdims=True)
        m1 = jnp.max(p, axis=-1, keepdims=True)
        i1 = jnp.min(jnp.where(p == m1, col, EPAD), axis=-1, keepdims=True)
        p2 = jnp.where(col == i1, -1.0, p)
        m2 = jnp.max(p2, axis=-1, keepdims=True)
        i2 = jnp.min(jnp.where(p2 == m2, col, EPAD), axis=-1, keepdims=True)
        sw = m1 + m2 + 1e-9
        cws[:] = (jnp.where(col == i1, m1, 0.0)
                  + jnp.where(col == i2, m2, 0.0)) / sw

    @pl.when(i > 0)
    def _expert_phase():
        e = (i - 1) // NQ
        qtr = (i - 1) % NQ
        hid = _gelu(_mm(h2s[:].astype(_bf16), w1_ref[0].astype(_bf16))
                    + b1_ref[0])
        ye = _mm(hid.astype(_bf16), w2_ref[0].astype(_bf16))
        ye = ye + jnp.where(qtr == 0, 1.0, 0.0) * eb2_ref[0]
        colh = jax.lax.broadcasted_iota(jnp.int32, cws.shape, 1)
        w = jnp.sum(jnp.where(colh == e, cws[:], 0.0), axis=-1, keepdims=True)
        o_ref[:] += w * ye


def _head_kern(x_ref, g_ref, b_ref, w_ref, hb_ref, o_ref):
    h = _ln(x_ref[:], g_ref[:], b_ref[:])
    o_ref[:] = _mm(h.astype(_bf16), w_ref[:].astype(_bf16)) + hb_ref[:]


def _row(v):
    return v.reshape(1, -1)


def kernel(pixel_values, params):
    b = pixel_values.shape[0]
    n = b * TP

    patches = (pixel_values.reshape(b, 3, 14, 16, 14, 16)
               .transpose(0, 2, 4, 1, 3, 5).reshape(b * 196, 768))
    emb = pl.pallas_call(
        _embed_kern,
        out_shape=jax.ShapeDtypeStruct((b * 196, D), f32),
    )(patches, params["patch_w"], _row(params["patch_b"]))
    emb = emb.reshape(b, 196, D)
    cls = jnp.broadcast_to(params["cls"], (b, 1, D))
    x = jnp.concatenate([cls, emb], axis=1) + params["pos"]
    x = jnp.pad(x, ((0, 0), (0, TP - T), (0, 0))).reshape(n, D)

    def cst(i):
        return (0, 0)

    wspec = pl.BlockSpec((D, D), cst)
    rspec = pl.BlockSpec((1, D), cst)
    ospec = pl.BlockSpec((n, D), cst)
    wscratch = [pltpu.VMEM((D, D), _bf16) for _ in range(4)]

    for lp in params["layers"]:
        bqkv = jnp.concatenate(
            [lp["q_b"], lp["k_b"], lp["v_b"]]).reshape(1, 3 * D)
        attn_in = (x, _row(lp["ln1_g"]), _row(lp["ln1_b"]),
                   lp["q_w"], lp["k_w"], lp["v_w"], lp["o_w"],
                   bqkv, _row(lp["o_b"]))
        attn_specs = [ospec, rspec, rspec, wspec, wspec, wspec, wspec,
                      pl.BlockSpec((1, 3 * D), cst), rspec]

        if "gate_w" in lp:
            gw = jnp.pad(lp["gate_w"], ((0, 0), (0, EPAD - E)))

            def eidx(i):
                return jnp.maximum(i - 1, 0) // NQ

            def hidx(i):
                return jnp.maximum(i - 1, 0) % NQ

            x = pl.pallas_call(
                functools.partial(_moe_layer_kern, b),
                grid=(1 + NQ * E,),
                in_specs=attn_specs + [
                    rspec, rspec, pl.BlockSpec((D, EPAD), cst),
                    pl.BlockSpec((1, D, IQ), lambda i: (eidx(i), 0, hidx(i))),
                    pl.BlockSpec((1, 1, IQ), lambda i: (eidx(i), 0, hidx(i))),
                    pl.BlockSpec((1, IQ, D), lambda i: (eidx(i), hidx(i), 0)),
                    pl.BlockSpec((1, 1, D), lambda i: (eidx(i), 0, 0)),
                ],
                out_specs=ospec,
                out_shape=jax.ShapeDtypeStruct((n, D), f32),
                scratch_shapes=wscratch + [pltpu.VMEM((n, D), f32),
                                           pltpu.VMEM((n, EPAD), f32)],
            )(*attn_in, _row(lp["ln2_g"]), _row(lp["ln2_b"]), gw,
              lp["e_w1"], lp["e_b1"].reshape(E, 1, I),
              lp["e_w2"], lp["e_b2"].reshape(E, 1, D))
        else:
            def jidx(i):
                return jnp.maximum(i - 1, 0)

            x = pl.pallas_call(
                functools.partial(_dense_layer_kern, b),
                grid=(1 + NI,),
                in_specs=attn_specs + [
                    rspec, rspec,
                    pl.BlockSpec((D, IB), lambda i: (0, jidx(i))),
                    pl.BlockSpec((1, IB), lambda i: (0, jidx(i))),
                    pl.BlockSpec((IB, D), lambda i: (jidx(i), 0)),
                    rspec,
                ],
                out_specs=ospec,
                out_shape=jax.ShapeDtypeStruct((n, D), f32),
                scratch_shapes=wscratch + [pltpu.VMEM((n, D), f32)],
            )(*attn_in, _row(lp["ln2_g"]), _row(lp["ln2_b"]),
              lp["w1"], _row(lp["b1"]), lp["w2"], _row(lp["b2"]))

    cls_tok = x.reshape(b, TP, D)[:, 0, :]
    cls_tok = jnp.pad(cls_tok, ((0, 8 - b), (0, 0)))
    hw = jnp.pad(params["head_w"], ((0, 0), (0, 128 - NC)))
    hb = jnp.pad(params["head_b"], (0, 128 - NC)).reshape(1, 128)
    logits = pl.pallas_call(
        _head_kern,
        out_shape=jax.ShapeDtypeStruct((8, 128), f32),
    )(cls_tok, _row(params["ln_f_g"]), _row(params["ln_f_b"]), hw, hb)
    return logits[:b, :NC]


# speedup vs baseline: 2.3512x; 1.0159x over previous
import functools
import math

import jax
import jax.numpy as jnp
from jax.experimental import pallas as pl
from jax.experimental.pallas import tpu as pltpu

D = 768
I = 3072
H = 12
DH = 64
E = 8
NC = 100
T = 197
TP = 208
EPAD = 128
NI = 2
IB = I // NI
NQ = 2
IQ = I // NQ

_bf16 = jnp.bfloat16
f32 = jnp.float32


def _mm(a, b):
    return jax.lax.dot_general(
        a, b, (((a.ndim - 1,), (0,)), ((), ())),
        preferred_element_type=jnp.float32)


def _ln(x, g, b):
    m = jnp.mean(x, axis=-1, keepdims=True)
    v = jnp.mean((x - m) ** 2, axis=-1, keepdims=True)
    return (x - m) / jnp.sqrt(v + 1e-12) * g + b


def _gelu(x):
    return 0.5 * x * (1.0 + jax.lax.erf(x / math.sqrt(2.0)))


def _embed_kern(p_ref, w_ref, b_ref, o_ref):
    o_ref[:] = _mm(p_ref[:].astype(_bf16), w_ref[:].astype(_bf16)) + b_ref[:]


def _attn_x1(nb, x_ref, g_ref, b_ref, qw_ref, kw_ref, vw_ref, ow_ref,
             bias_ref, ob_ref, qs, ks, vs, os):
    qs[:] = qw_ref[:].astype(_bf16)
    ks[:] = kw_ref[:].astype(_bf16)
    vs[:] = vw_ref[:].astype(_bf16)
    os[:] = ow_ref[:].astype(_bf16)
    x = x_ref[:]
    h = _ln(x, g_ref[:], b_ref[:]).astype(_bf16)
    q = (_mm(h, qs[:]) + bias_ref[:, 0:D]).astype(_bf16)
    k = (_mm(h, ks[:]) + bias_ref[:, D:2 * D]).astype(_bf16)
    v = (_mm(h, vs[:]) + bias_ref[:, 2 * D:3 * D]).astype(_bf16)
    valid = jax.lax.broadcasted_iota(jnp.int32, (TP, TP), 1) < T
    rows = []
    for bi in range(nb):
        r0, r1 = bi * TP, (bi + 1) * TP
        parts = []
        for j in range(H):
            c0, c1 = j * DH, (j + 1) * DH
            qj = q[r0:r1, c0:c1]
            kj = k[r0:r1, c0:c1]
            vj = v[r0:r1, c0:c1]
            s = jax.lax.dot_general(
                qj, kj, (((1,), (1,)), ((), ())),
                preferred_element_type=f32)
            s = s * (1.0 / math.sqrt(DH))
            s = jnp.where(valid, s, -1e30)
            m = jnp.max(s, axis=-1, keepdims=True)
            p = jnp.exp(s - m)
            a = p / jnp.sum(p, axis=-1, keepdims=True)
            parts.append(_mm(a.astype(_bf16), vj))
        rows.append(jnp.concatenate(parts, axis=1))
    ctx = jnp.concatenate(rows, axis=0)
    return _mm(ctx.astype(_bf16), os[:]) + ob_ref[:] + x


def _dense_layer_kern(nb, x_ref, g_ref, b_ref, qw_ref, kw_ref, vw_ref,
                      ow_ref, bias_ref, ob_ref, g2_ref, b2_ref,
                      w1_ref, b1_ref, w2_ref, fb2_ref,
                      o_ref, qs, ks, vs, os, h2s):
    i = pl.program_id(0)

    @pl.when(i == 0)
    def _attn_phase():
        x1 = _attn_x1(nb, x_ref, g_ref, b_ref, qw_ref, kw_ref, vw_ref,
                      ow_ref, bias_ref, ob_ref, qs, ks, vs, os)
        h2s[:] = _ln(x1, g2_ref[:], b2_ref[:])
        o_ref[:] = x1 + fb2_ref[:]

    @pl.when(i > 0)
    def _ffn_phase():
        hid = _gelu(_mm(h2s[:].astype(_bf16), w1_ref[:].astype(_bf16))
                    + b1_ref[:])
        o_ref[:] += _mm(hid.astype(_bf16), w2_ref[:].astype(_bf16))


def _moe_layer_kern(nb, x_ref, g_ref, b_ref, qw_ref, kw_ref, vw_ref,
                    ow_ref, bias_ref, ob_ref, g2_ref, b2_ref, gw_ref,
                    w1_ref, b1_ref, w2_ref, eb2_ref,
                    o_ref, qs, ks, vs, os, h2s, cws):
    i = pl.program_id(0)

    @pl.when(i == 0)
    def _attn_phase():
        x1 = _attn_x1(nb, x_ref, g_ref, b_ref, qw_ref, kw_ref, vw_ref,
                      ow_ref, bias_ref, ob_ref, qs, ks, vs, os)
        h2 = _ln(x1, g2_ref[:], b2_ref[:])
        h2s[:] = h2
        o_ref[:] = x1
        n = x1.shape[0]
        logits = _mm(h2.astype(_bf16), gw_ref[:].astype(_bf16))
        col = jax.lax.broadcasted_iota(jnp.int32, (n, EPAD), 1)
        logits = jnp.where(col < E, logits, -1e30)
        m = jnp.max(logits, axis=-1, keepdims=True)
        p = jnp.exp(logits - m)
        p = p / jnp.sum(p, axis=-1, keepdims=True)
        m1 = jnp.max(p, axis=-1, keepdims=True)
        i1 = jnp.min(jnp.where(p == m1, col, EPAD), axis=-1, keepdims=True)
        p2 = jnp.where(col == i1, -1.0, p)
        m2 = jnp.max(p2, axis=-1, keepdims=True)
        i2 = jnp.min(jnp.where(p2 == m2, col, EPAD), axis=-1, keepdims=True)
        sw = m1 + m2 + 1e-9
        cws[:] = (jnp.where(col == i1, m1, 0.0)
                  + jnp.where(col == i2, m2, 0.0)) / sw

    @pl.when(i > 0)
    def _expert_phase():
        e = (i - 1) // NQ
        qtr = (i - 1) % NQ
        hid = _gelu(_mm(h2s[:].astype(_bf16), w1_ref[0].astype(_bf16))
                    + b1_ref[0])
        ye = _mm(hid.astype(_bf16), w2_ref[0].astype(_bf16))
        ye = ye + jnp.where(qtr == 0, 1.0, 0.0) * eb2_ref[0]
        colh = jax.lax.broadcasted_iota(jnp.int32, cws.shape, 1)
        w = jnp.sum(jnp.where(colh == e, cws[:], 0.0), axis=-1, keepdims=True)
        o_ref[:] += w * ye


def _head_kern(x_ref, g_ref, b_ref, w_ref, hb_ref, o_ref):
    h = _ln(x_ref[:], g_ref[:], b_ref[:])
    o_ref[:] = _mm(h.astype(_bf16), w_ref[:].astype(_bf16)) + hb_ref[:]


def _row(v):
    return v.reshape(1, -1)


def kernel(pixel_values, params):
    b = pixel_values.shape[0]
    n = b * TP

    patches = (pixel_values.reshape(b, 3, 14, 16, 14, 16)
               .transpose(0, 2, 4, 1, 3, 5).reshape(b * 196, 768))
    emb = pl.pallas_call(
        _embed_kern,
        out_shape=jax.ShapeDtypeStruct((b * 196, D), f32),
    )(patches, params["patch_w"], _row(params["patch_b"]))
    emb = emb.reshape(b, 196, D)
    cls = jnp.broadcast_to(params["cls"], (b, 1, D))
    x = jnp.concatenate([cls, emb], axis=1) + params["pos"]
    x = jnp.pad(x, ((0, 0), (0, TP - T), (0, 0))).reshape(n, D)

    def cst(i):
        return (0, 0)

    wspec = pl.BlockSpec((D, D), cst)
    rspec = pl.BlockSpec((1, D), cst)
    ospec = pl.BlockSpec((n, D), cst)
    wscratch = [pltpu.VMEM((D, D), _bf16) for _ in range(4)]

    for lp in params["layers"]:
        bqkv = jnp.concatenate(
            [lp["q_b"], lp["k_b"], lp["v_b"]]).reshape(1, 3 * D)
        attn_in = (x, _row(lp["ln1_g"]), _row(lp["ln1_b"]),
                   lp["q_w"], lp["k_w"], lp["v_w"], lp["o_w"],
                   bqkv, _row(lp["o_b"]))
        attn_specs = [ospec, rspec, rspec, wspec, wspec, wspec, wspec,
                      pl.BlockSpec((1, 3 * D), cst), rspec]

        if "gate_w" in lp:
            gw = jnp.pad(lp["gate_w"], ((0, 0), (0, EPAD - E)))

            def eidx(i):
                return jnp.maximum(i - 1, 0) // NQ

            def hidx(i):
                return jnp.maximum(i - 1, 0) % NQ

            x = pl.pallas_call(
                functools.partial(_moe_layer_kern, b),
                grid=(1 + NQ * E,),
                in_specs=attn_specs + [
                    rspec, rspec, pl.BlockSpec((D, EPAD), cst),
                    pl.BlockSpec((1, D, IQ), lambda i: (eidx(i), 0, hidx(i))),
                    pl.BlockSpec((1, 1, IQ), lambda i: (eidx(i), 0, hidx(i))),
                    pl.BlockSpec((1, IQ, D), lambda i: (eidx(i), hidx(i), 0)),
                    pl.BlockSpec((1, 1, D), lambda i: (eidx(i), 0, 0)),
                ],
                out_specs=ospec,
                out_shape=jax.ShapeDtypeStruct((n, D), f32),
                scratch_shapes=wscratch + [pltpu.VMEM((n, D), f32),
                                           pltpu.VMEM((n, EPAD), f32)],
            )(*attn_in, _row(lp["ln2_g"]), _row(lp["ln2_b"]), gw,
              lp["e_w1"], lp["e_b1"].reshape(E, 1, I),
              lp["e_w2"], lp["e_b2"].reshape(E, 1, D))
        else:
            def jidx(i):
                return jnp.maximum(i - 1, 0)

            x = pl.pallas_call(
                functools.partial(_dense_layer_kern, b),
                grid=(1 + NI,),
                in_specs=attn_specs + [
                    rspec, rspec,
                    pl.BlockSpec((D, IB), lambda i: (0, jidx(i))),
                    pl.BlockSpec((1, IB), lambda i: (0, jidx(i))),
                    pl.BlockSpec((IB, D), lambda i: (jidx(i), 0)),
                    rspec,
                ],
                out_specs=ospec,
                out_shape=jax.ShapeDtypeStruct((n, D), f32),
                scratch_shapes=wscratch + [pltpu.VMEM((n, D), f32)],
            )(*attn_in, _row(lp["ln2_g"]), _row(lp["ln2_b"]),
              lp["w1"], _row(lp["b1"]), lp["w2"], _row(lp["b2"]))

    cls_tok = x.reshape(b, TP, D)[:, 0, :]
    cls_tok = jnp.pad(cls_tok, ((0, 8 - b), (0, 0)))
    hw = jnp.pad(params["head_w"], ((0, 0), (0, 128 - NC)))
    hb = jnp.pad(params["head_b"], (0, 128 - NC)).reshape(1, 128)
    logits = pl.pallas_call(
        _head_kern,
        out_shape=jax.ShapeDtypeStruct((8, 128), f32),
    )(cls_tok, _row(params["ln_f_g"]), _row(params["ln_f_b"]), hw, hb)
    return logits[:b, :NC]
